# baseline (speedup 1.0000x reference)
import jax
import jax.numpy as jnp
from jax import lax
from jax.experimental import pallas as pl
from jax.experimental.pallas import tpu as pltpu
from jax.experimental.pallas import tpu_sc as plsc

D = 1024
NC, NS = 2, 16
NW = NC * NS
CH = 16


def _sc_body(emb_hbm, pos_hbm, tim_hbm, flg_hbm, pos_tab_hbm, tim_tab_hbm,
             out_hbm, pos_idx, tim_idx, pos_buf, tim_buf,
             sem_i, sem_e, sem_f, sem_g, sem_w):
    T = pos_hbm.shape[0]
    tpw = T // NW
    wid = lax.axis_index("s") * NC + lax.axis_index("c")
    base = wid * tpw

    ci0 = pltpu.async_copy(pos_hbm.at[pl.ds(base, tpw)], pos_idx, sem_i)
    ci1 = pltpu.async_copy(tim_hbm.at[pl.ds(base, tpw)], tim_idx, sem_i)
    ce = pltpu.async_copy(emb_hbm.at[pl.ds(base, tpw)],
                          out_hbm.at[pl.ds(base, tpw), pl.ds(0, D)], sem_e)
    cf = pltpu.async_copy(flg_hbm.at[pl.ds(base, tpw)],
                          out_hbm.at[pl.ds(base, tpw), pl.ds(3 * D, 1)],
                          sem_f)
    ci0.wait()
    ci1.wait()

    def chunk(i, _):
        tok = base + i * CH
        off = i * CH
        gp = pltpu.async_copy(
            pos_tab_hbm.at[pos_idx.at[pl.ds(off, CH)]], pos_buf, sem_g)
        gt = pltpu.async_copy(
            tim_tab_hbm.at[tim_idx.at[pl.ds(off, CH)]], tim_buf, sem_g)
        gp.wait()
        gt.wait()
        wp = pltpu.async_copy(
            pos_buf, out_hbm.at[pl.ds(tok, CH), pl.ds(D, D)], sem_w)
        wt = pltpu.async_copy(
            tim_buf, out_hbm.at[pl.ds(tok, CH), pl.ds(2 * D, D)], sem_w)
        wp.wait()
        wt.wait()
        return ()

    lax.fori_loop(0, tpw // CH, chunk, ())
    ce.wait()
    cf.wait()


def kernel(embeddings, positions, times, source_flags, pos_table, time_table):
    B, S, Dm = embeddings.shape
    T = B * S
    tpw = T // NW
    emb = embeddings.reshape(T, Dm)
    pos = positions.reshape(T).astype(jnp.int32)
    tim = times.reshape(T).astype(jnp.int32)
    flg = source_flags.reshape(T, 1).astype(jnp.float32)
    mesh = plsc.VectorSubcoreMesh(
        core_axis_name="c", subcore_axis_name="s",
        num_cores=NC, num_subcores=NS)
    out = pl.kernel(
        _sc_body,
        out_type=jax.ShapeDtypeStruct((T, 3 * Dm + 1), jnp.float32),
        mesh=mesh,
        scratch_types=[
            pltpu.VMEM((tpw,), jnp.int32),
            pltpu.VMEM((tpw,), jnp.int32),
            pltpu.VMEM((CH, Dm), jnp.float32),
            pltpu.VMEM((CH, Dm), jnp.float32),
            pltpu.SemaphoreType.DMA,
            pltpu.SemaphoreType.DMA,
            pltpu.SemaphoreType.DMA,
            pltpu.SemaphoreType.DMA,
            pltpu.SemaphoreType.DMA,
        ],
    )(emb, pos, tim, flg, pos_table, time_table)
    return out.reshape(B, S, 3 * Dm + 1)


# speedup vs baseline: 7.3383x; 7.3383x over previous
import jax
import jax.numpy as jnp
from jax import lax
from jax.experimental import pallas as pl
from jax.experimental.pallas import tpu as pltpu
from jax.experimental.pallas import tpu_sc as plsc

D = 1024
NC, NS = 2, 16
NW = NC * NS
CH = 16


def _sc_body(emb_hbm, pos_hbm, tim_hbm, flg_hbm, pos_tab_hbm, tim_tab_hbm,
             out_hbm, pos_idx, tim_idx, flg_v, pos_buf, tim_buf, emb_buf,
             sem_i, sem_f, sem_g, sem_w):
    T = pos_hbm.shape[0]
    tpw = T // NW
    wid = lax.axis_index("s") * NC + lax.axis_index("c")
    base = wid * tpw

    ci0 = pltpu.async_copy(pos_hbm.at[pl.ds(base, tpw)], pos_idx, sem_i)
    ci1 = pltpu.async_copy(tim_hbm.at[pl.ds(base, tpw)], tim_idx, sem_i)
    ci2 = pltpu.async_copy(flg_hbm.at[pl.ds(base, tpw)], flg_v, sem_i)
    ci0.wait()
    ci1.wait()
    ci2.wait()
    cf = pltpu.async_copy(flg_v, out_hbm.at[pl.ds(base, tpw), pl.ds(3 * D, 1)],
                          sem_f)

    def chunk(i, _):
        tok = base + i * CH
        off = i * CH
        gp = pltpu.async_copy(
            pos_tab_hbm.at[pos_idx.at[pl.ds(off, CH)]], pos_buf, sem_g)
        gt = pltpu.async_copy(
            tim_tab_hbm.at[tim_idx.at[pl.ds(off, CH)]], tim_buf, sem_g)
        ge = pltpu.async_copy(emb_hbm.at[pl.ds(tok, CH)], emb_buf, sem_g)
        gp.wait()
        gt.wait()
        ge.wait()
        we = pltpu.async_copy(
            emb_buf, out_hbm.at[pl.ds(tok, CH), pl.ds(0, D)], sem_w)
        wp = pltpu.async_copy(
            pos_buf, out_hbm.at[pl.ds(tok, CH), pl.ds(D, D)], sem_w)
        wt = pltpu.async_copy(
            tim_buf, out_hbm.at[pl.ds(tok, CH), pl.ds(2 * D, D)], sem_w)
        we.wait()
        wp.wait()
        wt.wait()
        return ()

    lax.fori_loop(0, tpw // CH, chunk, ())
    cf.wait()


def kernel(embeddings, positions, times, source_flags, pos_table, time_table):
    B, S, Dm = embeddings.shape
    T = B * S
    tpw = T // NW
    emb = embeddings.reshape(T, Dm)
    pos = positions.reshape(T).astype(jnp.int32)
    tim = times.reshape(T).astype(jnp.int32)
    flg = source_flags.reshape(T, 1).astype(jnp.float32)
    mesh = plsc.VectorSubcoreMesh(
        core_axis_name="c", subcore_axis_name="s",
        num_cores=NC, num_subcores=NS)
    out = pl.kernel(
        _sc_body,
        out_type=jax.ShapeDtypeStruct((T, 3 * Dm + 1), jnp.float32),
        mesh=mesh,
        scratch_types=[
            pltpu.VMEM((tpw,), jnp.int32),
            pltpu.VMEM((tpw,), jnp.int32),
            pltpu.VMEM((tpw, 1), jnp.float32),
            pltpu.VMEM((CH, Dm), jnp.float32),
            pltpu.VMEM((CH, Dm), jnp.float32),
            pltpu.VMEM((CH, Dm), jnp.float32),
            pltpu.SemaphoreType.DMA,
            pltpu.SemaphoreType.DMA,
            pltpu.SemaphoreType.DMA,
            pltpu.SemaphoreType.DMA,
        ],
    )(emb, pos, tim, flg, pos_table, time_table)
    return out.reshape(B, S, 3 * Dm + 1)
